# Initial kernel scaffold; baseline (speedup 1.0000x reference)
import jax
import jax.numpy as jnp
from jax.experimental import pallas as pl


def kernel(feats_a, feats_b, batch_index, index_map, W_emb_a, b_emb_a, W_emb_b, b_emb_b, W1, b1, W2, b2, W_head, b_head, W_aux, b_aux):
    raise NotImplementedError("write your pallas kernel here")


# speedup vs baseline: 1.9342x; 1.9342x over previous
import functools

import jax
import jax.numpy as jnp
from jax import lax
from jax.experimental import pallas as pl
from jax.experimental.pallas import tpu as pltpu
from jax.experimental.pallas import tpu_sc as plsc

N = 16384
N_A = 8192
D_PAD = 64
D_MODEL = 512
D_FF = 1024
N_ACT = 32
B = 16
BLK = 256
GRID = N // BLK
A_BLOCKS = N_A // BLK

NC = 2
NS = 16
NW = NC * NS
ROWS_W = N // NW
CH = 128
NCH = ROWS_W // CH

_sc_mesh = functools.partial(
    plsc.VectorSubcoreMesh, core_axis_name="c", subcore_axis_name="s")
_sc_params = pltpu.CompilerParams(use_tc_tiling_on_sc=False)


def _hist_body(idx_hbm, out_hbm, table, idx_v, ones_v, zbuf):
    cid = lax.axis_index("c")
    sid = lax.axis_index("s")
    wid = cid * NS + sid

    for k in range(zbuf.shape[0] // 16):
        zbuf[pl.ds(k * 16, 16)] = jnp.zeros((16,), jnp.float32)
    pltpu.sync_copy(zbuf, table.at[pl.ds(sid * (N // NS), N // NS)])
    for k in range(CH // 16):
        ones_v[pl.ds(k * 16, 16)] = jnp.ones((16,), jnp.float32)
    plsc.subcore_barrier()

    pltpu.sync_copy(idx_hbm.at[pl.ds(wid * NCH, NCH)], idx_v)
    for j in range(NCH):
        pltpu.sync_copy(ones_v, table.at[idx_v.at[j]], add=True)
    plsc.subcore_barrier()

    pltpu.sync_copy(table.at[pl.ds(sid * (N // NS), N // NS)], zbuf)
    pltpu.sync_copy(zbuf, out_hbm.at[cid, pl.ds(sid * (N // NS), N // NS)])


def _hist(idx2d):
    k = pl.kernel(
        _hist_body,
        out_type=jax.ShapeDtypeStruct((NC, N), jnp.float32),
        mesh=_sc_mesh(),
        compiler_params=_sc_params,
        scratch_types=[
            pltpu.VMEM_SHARED((N,), jnp.float32),
            pltpu.VMEM((NCH, CH), jnp.int32),
            pltpu.VMEM((CH,), jnp.float32),
            pltpu.VMEM((N // NS,), jnp.float32),
        ],
    )
    return k(idx2d)


def _gather_body(z_hbm, idx_hbm, out_hbm, idx_v, rows_v, sem):
    cid = lax.axis_index("c")
    sid = lax.axis_index("s")
    wid = cid * NS + sid

    pltpu.sync_copy(idx_hbm.at[pl.ds(wid * NCH, NCH)], idx_v)
    cps = []
    for j in range(NCH):
        cps.append(pltpu.async_copy(
            z_hbm.at[idx_v.at[j]], rows_v.at[pl.ds(j * CH, CH)], sem))
    for cp in cps:
        cp.wait()
    pltpu.sync_copy(rows_v, out_hbm.at[pl.ds(wid * ROWS_W, ROWS_W)])


def _gather(z, idx2d):
    k = pl.kernel(
        _gather_body,
        out_type=jax.ShapeDtypeStruct((N, N_ACT), jnp.float32),
        mesh=_sc_mesh(),
        compiler_params=_sc_params,
        scratch_types=[
            pltpu.VMEM((NCH, CH), jnp.int32),
            pltpu.VMEM((ROWS_W, N_ACT), jnp.float32),
            pltpu.SemaphoreType.DMA,
        ],
    )
    return k(z, idx2d)


def _dense_body(f_ref, wemb_ref, bemb_ref, w1_ref, b1_ref, w2_ref, b2_ref,
                wh_ref, bh_ref, wa_ref, cnta_ref, cntb_ref, bi_ref, baux_ref,
                logits_ref, aux_ref, accs_ref, accc_ref):
    i = pl.program_id(0)
    x0 = jnp.dot(f_ref[...], wemb_ref[0],
                 preferred_element_type=jnp.float32) + bemb_ref[0]
    h = jnp.dot(x0, w1_ref[...], preferred_element_type=jnp.float32)
    h = jnp.maximum(h + b1_ref[0], 0.0)
    y = x0 + jnp.dot(h, w2_ref[...],
                     preferred_element_type=jnp.float32) + b2_ref[...]
    logits_ref[...] = jnp.dot(y, wh_ref[...],
                              preferred_element_type=jnp.float32) + bh_ref[...]
    v = jnp.dot(y, wa_ref[...], preferred_element_type=jnp.float32)
    cnt = cnta_ref[...] + cntb_ref[...]
    seg_ids = lax.broadcasted_iota(jnp.int32, (BLK, B), 1)
    eq = (bi_ref[...] == seg_ids).astype(jnp.float32)
    ps = jnp.sum(eq * (cnt * v), axis=0, keepdims=True)
    pc = jnp.sum(eq * cnt, axis=0, keepdims=True)

    @pl.when(i == 0)
    def _():
        accs_ref[...] = jnp.zeros_like(accs_ref)
        accc_ref[...] = jnp.zeros_like(accc_ref)

    accs_ref[...] += ps
    accc_ref[...] += pc

    @pl.when(i == GRID - 1)
    def _():
        aux_ref[...] = (accs_ref[...] / jnp.maximum(accc_ref[...], 1.0)
                        + baux_ref[...])


def _dense(feats_pad, wemb, bemb, w1x, b1s, w2, b2, wh, bh, wa,
           cnt_a, cnt_b, bi_col, baux):
    return pl.pallas_call(
        _dense_body,
        grid=(GRID,),
        in_specs=[
            pl.BlockSpec((BLK, D_PAD), lambda i: (i, 0)),
            pl.BlockSpec((1, D_PAD, D_MODEL), lambda i: (i // A_BLOCKS, 0, 0)),
            pl.BlockSpec((1, 1, D_MODEL), lambda i: (i // A_BLOCKS, 0, 0)),
            pl.BlockSpec((D_MODEL, D_FF), lambda i: (0, 0)),
            pl.BlockSpec((1, 1, D_FF), lambda i: (i // A_BLOCKS, 0, 0)),
            pl.BlockSpec((D_FF, D_MODEL), lambda i: (0, 0)),
            pl.BlockSpec((1, D_MODEL), lambda i: (0, 0)),
            pl.BlockSpec((D_MODEL, N_ACT), lambda i: (0, 0)),
            pl.BlockSpec((1, N_ACT), lambda i: (0, 0)),
            pl.BlockSpec((D_MODEL, 1), lambda i: (0, 0)),
            pl.BlockSpec((BLK, 1), lambda i: (i, 0)),
            pl.BlockSpec((BLK, 1), lambda i: (i, 0)),
            pl.BlockSpec((BLK, 1), lambda i: (i, 0)),
            pl.BlockSpec((1, 1), lambda i: (0, 0)),
        ],
        out_specs=[
            pl.BlockSpec((BLK, N_ACT), lambda i: (i, 0)),
            pl.BlockSpec((1, B), lambda i: (0, 0)),
        ],
        out_shape=[
            jax.ShapeDtypeStruct((N, N_ACT), jnp.float32),
            jax.ShapeDtypeStruct((1, B), jnp.float32),
        ],
        scratch_shapes=[
            pltpu.VMEM((1, B), jnp.float32),
            pltpu.VMEM((1, B), jnp.float32),
        ],
        compiler_params=pltpu.CompilerParams(
            dimension_semantics=("arbitrary",)),
    )(feats_pad, wemb, bemb, w1x, b1s, w2, b2, wh, bh, wa,
      cnt_a, cnt_b, bi_col, baux)


BLK2 = 1024
GRID2 = N // BLK2


def _bcast_body(logits_ref, aux_ref, bi_ref, z_ref):
    seg_ids = lax.broadcasted_iota(jnp.int32, (BLK2, B), 1)
    eq = (bi_ref[...] == seg_ids).astype(jnp.float32)
    auxrow = jnp.sum(eq * aux_ref[...], axis=1, keepdims=True)
    z_ref[...] = logits_ref[...] + auxrow


def _bcast(logits, aux, bi_col):
    return pl.pallas_call(
        _bcast_body,
        grid=(GRID2,),
        in_specs=[
            pl.BlockSpec((BLK2, N_ACT), lambda i: (i, 0)),
            pl.BlockSpec((1, B), lambda i: (0, 0)),
            pl.BlockSpec((BLK2, 1), lambda i: (i, 0)),
        ],
        out_specs=pl.BlockSpec((BLK2, N_ACT), lambda i: (i, 0)),
        out_shape=jax.ShapeDtypeStruct((N, N_ACT), jnp.float32),
        compiler_params=pltpu.CompilerParams(
            dimension_semantics=("arbitrary",)),
    )(logits, aux, bi_col)


def kernel(feats_a, feats_b, batch_index, index_map,
           W_emb_a, b_emb_a, W_emb_b, b_emb_b,
           W1, b1, W2, b2, W_head, b_head, W_aux, b_aux):
    feats_pad = jnp.concatenate(
        [feats_a, jnp.pad(feats_b, ((0, 0), (0, D_PAD - feats_b.shape[1])))],
        axis=0)
    wemb = jnp.stack(
        [W_emb_a, jnp.pad(W_emb_b, ((0, D_PAD - W_emb_b.shape[0]), (0, 0)))])
    bemb = jnp.stack([b_emb_a, b_emb_b]).reshape(2, 1, D_MODEL)
    w1x = W1[:D_MODEL]
    b1s = jnp.stack([b1, b1 + W1[D_MODEL]]).reshape(2, 1, D_FF)
    b2r = b2.reshape(1, D_MODEL)
    bhr = b_head.reshape(1, N_ACT)
    bauxr = b_aux.reshape(1, 1)
    idx2d = index_map.reshape(CH, CH)
    bi_col = batch_index.reshape(N, 1)

    cnt = _hist(idx2d)
    logits, aux = _dense(feats_pad, wemb, bemb, w1x, b1s, W2, b2r,
                         W_head, bhr, W_aux,
                         cnt[0].reshape(N, 1), cnt[1].reshape(N, 1),
                         bi_col, bauxr)
    z = _bcast(logits, aux, bi_col)
    return _gather(z, idx2d)
